# baseline (speedup 1.0000x reference)
import jax
import jax.numpy as jnp
from jax import lax
from jax.experimental import pallas as pl
from jax.experimental.pallas import tpu as pltpu

EMB = 256
HID = 256
NREL = 6
N = 512

_NEG = -9e15


def _net_kernel(feat_ref, adj_ref, wfc1_ref, bfc1_ref,
                w0_ref, as0_ref, ad0_ref, c0_ref,
                w1_ref, as1_ref, ad1_ref, c1_ref,
                out_ref, w0a_ref, w1a_ref):
    @pl.when(pl.program_id(0) == 0)
    def _build_augmented():
        for w_ref, as_ref, ad_ref, wa_ref in (
                (w0_ref, as0_ref, ad0_ref, w0a_ref),
                (w1_ref, as1_ref, ad1_ref, w1a_ref)):
            w = w_ref[...]
            wa_s = lax.dot_general(w, as_ref[...], (((1,), (1,)), ((), ())),
                                   preferred_element_type=jnp.float32)
            wa_d = lax.dot_general(w, ad_ref[...], (((1,), (1,)), ((), ())),
                                   preferred_element_type=jnp.float32)
            wa_ref[:, :HID] = w
            wa_ref[:, HID:] = jnp.concatenate(
                [wa_s, wa_d, jnp.zeros((HID, 126), jnp.float32)], axis=1)
    feat = feat_ref[0]
    adj = adj_ref[0]
    mask = adj > 0
    adj_bf = adj.astype(jnp.bfloat16)

    H = jnp.dot(feat, wfc1_ref[...], preferred_element_type=jnp.float32)
    H = jax.nn.relu(H + bfc1_ref[...])

    for wa_ref, c_ref in ((w0a_ref, c0_ref), (w1a_ref, c1_ref)):
        WhA = jnp.dot(H, wa_ref[...], preferred_element_type=jnp.float32)
        Wh = WhA[:, :HID]
        s_src = WhA[:, HID:HID + 1]
        s_dst = WhA[:, HID + 1:HID + 2]

        rel = jnp.full((N, N), c_ref[0, 1].astype(jnp.bfloat16),
                       dtype=jnp.bfloat16)
        for r in range(2, NREL):
            rel = jnp.where(adj_bf == r,
                            c_ref[0, r].astype(jnp.bfloat16), rel)

        e = (s_src + s_dst.reshape(1, N)) + rel.astype(jnp.float32)
        e = jnp.maximum(e, 0.2 * e)
        e = jnp.where(mask, e, _NEG)
        m = jnp.max(e, axis=1, keepdims=True)
        p = jnp.exp(e - m)
        s = jnp.sum(p, axis=1, keepdims=True)
        inv = jnp.where(m > -8e15, 1.0 / s, 0.0)

        out = jnp.dot(p, Wh, preferred_element_type=jnp.float32) * inv
        out = jnp.where(out > 0, out, jnp.exp(out) - 1.0)
        H = out + H

    out_ref[0, :, :HID] = H
    out_ref[0, :, HID:] = feat


@jax.jit
def kernel(utterance_features, semantic_adj, q_type, pos,
           W_fc1, b_fc1,
           W_gat0, a_src0, a_dst0, rel_bias0,
           W_gat1, a_src1, a_dst1, rel_bias1):
    del q_type, pos
    B = utterance_features.shape[0]

    row = lambda v: v.reshape(1, -1)

    grid = (B,)
    in_specs = [
            pl.BlockSpec((1, N, EMB), lambda b: (b, 0, 0)),
            pl.BlockSpec((1, N, N), lambda b: (b, 0, 0)),
            pl.BlockSpec((EMB, HID), lambda b: (0, 0)),
            pl.BlockSpec((1, HID), lambda b: (0, 0)),
            pl.BlockSpec((HID, HID), lambda b: (0, 0)),
            pl.BlockSpec((1, HID), lambda b: (0, 0)),
            pl.BlockSpec((1, HID), lambda b: (0, 0)),
            pl.BlockSpec((1, NREL), lambda b: (0, 0)),
            pl.BlockSpec((HID, HID), lambda b: (0, 0)),
            pl.BlockSpec((1, HID), lambda b: (0, 0)),
            pl.BlockSpec((1, HID), lambda b: (0, 0)),
            pl.BlockSpec((1, NREL), lambda b: (0, 0)),
    ]
    out_specs = pl.BlockSpec((1, N, HID + EMB), lambda b: (b, 0, 0))

    return pl.pallas_call(
        _net_kernel,
        grid=grid,
        in_specs=in_specs,
        out_specs=out_specs,
        out_shape=jax.ShapeDtypeStruct((B, N, HID + EMB), jnp.float32),
        scratch_shapes=[pltpu.VMEM((HID, HID + 128), jnp.float32),
                        pltpu.VMEM((HID, HID + 128), jnp.float32)],
    )(utterance_features, semantic_adj,
      W_fc1, row(b_fc1),
      W_gat0, row(a_src0), row(a_dst0), row(rel_bias0),
      W_gat1, row(a_src1), row(a_dst1), row(rel_bias1))


# speedup vs baseline: 1.0057x; 1.0057x over previous
import jax
import jax.numpy as jnp
from jax import lax
from jax.experimental import pallas as pl
from jax.experimental.pallas import tpu as pltpu

EMB = 256
HID = 256
NREL = 6
N = 512

_NEG = -9e15


def _net_kernel(feat_ref, adj_ref, wfc1_ref, bfc1_ref,
                w0_ref, as0_ref, ad0_ref, c0_ref,
                w1_ref, as1_ref, ad1_ref, c1_ref,
                out_ref, w0a_ref, w1a_ref):
    @pl.when(pl.program_id(0) == 0)
    def _build_augmented():
        for w_ref, as_ref, ad_ref, wa_ref in (
                (w0_ref, as0_ref, ad0_ref, w0a_ref),
                (w1_ref, as1_ref, ad1_ref, w1a_ref)):
            w = w_ref[...]
            wa_s = lax.dot_general(w, as_ref[...], (((1,), (1,)), ((), ())),
                                   preferred_element_type=jnp.float32)
            wa_d = lax.dot_general(w, ad_ref[...], (((1,), (1,)), ((), ())),
                                   preferred_element_type=jnp.float32)
            wa_ref[...] = jnp.concatenate(
                [wa_s, wa_d, jnp.zeros((HID, 126), jnp.float32)], axis=1)
    feat = feat_ref[0]
    adj = adj_ref[0]
    mask = adj > 0
    adj_bf = adj.astype(jnp.bfloat16)

    H = jnp.dot(feat, wfc1_ref[...], preferred_element_type=jnp.float32)
    H = jax.nn.relu(H + bfc1_ref[...])

    for w_ref, wa_ref, c_ref in ((w0_ref, w0a_ref, c0_ref),
                                 (w1_ref, w1a_ref, c1_ref)):
        Wh = jnp.dot(H, w_ref[...], preferred_element_type=jnp.float32)
        S2 = jnp.dot(H, wa_ref[...], preferred_element_type=jnp.float32)
        s_src = S2[:, 0:1]
        s_dst = S2[:, 1:2]

        rel = jnp.full((N, N), c_ref[0, 1].astype(jnp.bfloat16),
                       dtype=jnp.bfloat16)
        for r in range(2, NREL):
            rel = jnp.where(adj_bf == r,
                            c_ref[0, r].astype(jnp.bfloat16), rel)

        e = (s_src + s_dst.reshape(1, N)) + rel.astype(jnp.float32)
        e = jnp.maximum(e, 0.2 * e)
        e = jnp.where(mask, e, _NEG)
        m = jnp.max(e, axis=1, keepdims=True)
        p = jnp.exp(e - m)
        s = jnp.sum(p, axis=1, keepdims=True)
        inv = jnp.where(m > -8e15, 1.0 / s, 0.0)

        out = jnp.dot(p, Wh, preferred_element_type=jnp.float32) * inv
        out = jnp.where(out > 0, out, jnp.exp(out) - 1.0)
        H = out + H

    out_ref[0, :, :HID] = H
    out_ref[0, :, HID:] = feat


@jax.jit
def kernel(utterance_features, semantic_adj, q_type, pos,
           W_fc1, b_fc1,
           W_gat0, a_src0, a_dst0, rel_bias0,
           W_gat1, a_src1, a_dst1, rel_bias1):
    del q_type, pos
    B = utterance_features.shape[0]

    row = lambda v: v.reshape(1, -1)

    grid = (B,)
    in_specs = [
            pl.BlockSpec((1, N, EMB), lambda b: (b, 0, 0)),
            pl.BlockSpec((1, N, N), lambda b: (b, 0, 0)),
            pl.BlockSpec((EMB, HID), lambda b: (0, 0)),
            pl.BlockSpec((1, HID), lambda b: (0, 0)),
            pl.BlockSpec((HID, HID), lambda b: (0, 0)),
            pl.BlockSpec((1, HID), lambda b: (0, 0)),
            pl.BlockSpec((1, HID), lambda b: (0, 0)),
            pl.BlockSpec((1, NREL), lambda b: (0, 0)),
            pl.BlockSpec((HID, HID), lambda b: (0, 0)),
            pl.BlockSpec((1, HID), lambda b: (0, 0)),
            pl.BlockSpec((1, HID), lambda b: (0, 0)),
            pl.BlockSpec((1, NREL), lambda b: (0, 0)),
    ]
    out_specs = pl.BlockSpec((1, N, HID + EMB), lambda b: (b, 0, 0))

    return pl.pallas_call(
        _net_kernel,
        grid=grid,
        in_specs=in_specs,
        out_specs=out_specs,
        out_shape=jax.ShapeDtypeStruct((B, N, HID + EMB), jnp.float32),
        scratch_shapes=[pltpu.VMEM((HID, 128), jnp.float32),
                        pltpu.VMEM((HID, 128), jnp.float32)],
    )(utterance_features, semantic_adj,
      W_fc1, row(b_fc1),
      W_gat0, row(a_src0), row(a_dst0), row(rel_bias0),
      W_gat1, row(a_src1), row(a_dst1), row(rel_bias1))


# speedup vs baseline: 1.0355x; 1.0296x over previous
import jax
import jax.numpy as jnp
from jax import lax
from jax.experimental import pallas as pl

EMB = 256
HID = 256
NREL = 6
N = 512

_NEG = -9e15


def _net_kernel(feat_ref, adj_ref, wfc1_ref, bfc1_ref,
                w0_ref, as0_ref, ad0_ref, c0_ref,
                w1_ref, as1_ref, ad1_ref, c1_ref,
                out_ref):
    feat = feat_ref[0]
    adj = adj_ref[0]
    mask = adj > 0
    adj_bf = adj.astype(jnp.bfloat16)

    H = jnp.dot(feat, wfc1_ref[...], preferred_element_type=jnp.float32)
    H = jax.nn.relu(H + bfc1_ref[...])

    for w_ref, as_ref, ad_ref, c_ref in (
            (w0_ref, as0_ref, ad0_ref, c0_ref),
            (w1_ref, as1_ref, ad1_ref, c1_ref)):
        Wh = jnp.dot(H, w_ref[...], preferred_element_type=jnp.float32)
        s_src = jnp.sum(Wh * as_ref[...], axis=1, keepdims=True)
        s_dst = jnp.sum(Wh * ad_ref[...], axis=1, keepdims=True)

        rel = jnp.full((N, N), c_ref[0, 1].astype(jnp.bfloat16),
                       dtype=jnp.bfloat16)
        for r in range(2, NREL):
            rel = jnp.where(adj_bf == r,
                            c_ref[0, r].astype(jnp.bfloat16), rel)

        e = (s_src + s_dst.reshape(1, N)) + rel.astype(jnp.float32)
        e = jnp.maximum(e, 0.2 * e)
        e = jnp.where(mask, e, _NEG)
        m = jnp.max(e, axis=1, keepdims=True)
        p = jnp.exp(e - m)
        s = jnp.sum(p, axis=1, keepdims=True)
        inv = jnp.where(m > -8e15, 1.0 / s, 0.0)

        out = jnp.dot(p.astype(jnp.bfloat16), Wh.astype(jnp.bfloat16),
                      preferred_element_type=jnp.float32) * inv
        out = jnp.where(out > 0, out, jnp.exp(out) - 1.0)
        H = out + H

    out_ref[0, :, :HID] = H
    out_ref[0, :, HID:] = feat


@jax.jit
def kernel(utterance_features, semantic_adj, q_type, pos,
           W_fc1, b_fc1,
           W_gat0, a_src0, a_dst0, rel_bias0,
           W_gat1, a_src1, a_dst1, rel_bias1):
    del q_type, pos
    B = utterance_features.shape[0]

    row = lambda v: v.reshape(1, -1)

    grid_spec = pl.GridSpec(
        grid=(B,),
        in_specs=[
            pl.BlockSpec((1, N, EMB), lambda b: (b, 0, 0)),
            pl.BlockSpec((1, N, N), lambda b: (b, 0, 0)),
            pl.BlockSpec((EMB, HID), lambda b: (0, 0)),
            pl.BlockSpec((1, HID), lambda b: (0, 0)),
            pl.BlockSpec((HID, HID), lambda b: (0, 0)),
            pl.BlockSpec((1, HID), lambda b: (0, 0)),
            pl.BlockSpec((1, HID), lambda b: (0, 0)),
            pl.BlockSpec((1, NREL), lambda b: (0, 0)),
            pl.BlockSpec((HID, HID), lambda b: (0, 0)),
            pl.BlockSpec((1, HID), lambda b: (0, 0)),
            pl.BlockSpec((1, HID), lambda b: (0, 0)),
            pl.BlockSpec((1, NREL), lambda b: (0, 0)),
        ],
        out_specs=pl.BlockSpec((1, N, HID + EMB), lambda b: (b, 0, 0)),
    )

    return pl.pallas_call(
        _net_kernel,
        grid_spec=grid_spec,
        out_shape=jax.ShapeDtypeStruct((B, N, HID + EMB), jnp.float32),
    )(utterance_features, semantic_adj,
      W_fc1, row(b_fc1),
      W_gat0, row(a_src0), row(a_dst0), row(rel_bias0),
      W_gat1, row(a_src1), row(a_dst1), row(rel_bias1))


# speedup vs baseline: 1.0951x; 1.0576x over previous
import jax
import jax.numpy as jnp
from jax import lax
from jax.experimental import pallas as pl

EMB = 256
HID = 256
NREL = 6
N = 512

_NEG = -9e15


def _net_kernel(feat_ref, adj_ref, wfc1_ref, bfc1_ref,
                w0_ref, as0_ref, ad0_ref, c0_ref,
                w1_ref, as1_ref, ad1_ref, c1_ref,
                out_ref):
    feat = feat_ref[0]
    adj = adj_ref[0]
    adj_bf = adj.astype(jnp.bfloat16)

    H = jnp.dot(feat, wfc1_ref[...], preferred_element_type=jnp.float32)
    H = jax.nn.relu(H + bfc1_ref[...])

    for w_ref, as_ref, ad_ref, c_ref in (
            (w0_ref, as0_ref, ad0_ref, c0_ref),
            (w1_ref, as1_ref, ad1_ref, c1_ref)):
        Wh = jnp.dot(H, w_ref[...], preferred_element_type=jnp.float32)
        s_src = jnp.sum(Wh * as_ref[...], axis=1, keepdims=True)
        s_dst = jnp.sum(Wh * ad_ref[...], axis=1, keepdims=True)

        rel = jnp.full((N, N), jnp.bfloat16(_NEG), dtype=jnp.bfloat16)
        for r in range(1, NREL):
            rel = jnp.where(adj_bf == r,
                            c_ref[0, r].astype(jnp.bfloat16), rel)

        e = (s_src + s_dst.reshape(1, N)) + rel.astype(jnp.float32)
        e = jnp.maximum(e, 0.2 * e)
        m = jnp.max(e, axis=1, keepdims=True)
        p = jnp.exp(e - m)
        s = jnp.sum(p, axis=1, keepdims=True)
        inv = jnp.where(m > -1e14, 1.0 / s, 0.0)

        out = jnp.dot(p, Wh, preferred_element_type=jnp.float32) * inv
        out = jnp.where(out > 0, out, jnp.exp(out) - 1.0)
        H = out + H

    out_ref[0, :, :HID] = H
    out_ref[0, :, HID:] = feat


@jax.jit
def kernel(utterance_features, semantic_adj, q_type, pos,
           W_fc1, b_fc1,
           W_gat0, a_src0, a_dst0, rel_bias0,
           W_gat1, a_src1, a_dst1, rel_bias1):
    del q_type, pos
    B = utterance_features.shape[0]

    row = lambda v: v.reshape(1, -1)

    grid_spec = pl.GridSpec(
        grid=(B,),
        in_specs=[
            pl.BlockSpec((1, N, EMB), lambda b: (b, 0, 0)),
            pl.BlockSpec((1, N, N), lambda b: (b, 0, 0)),
            pl.BlockSpec((EMB, HID), lambda b: (0, 0)),
            pl.BlockSpec((1, HID), lambda b: (0, 0)),
            pl.BlockSpec((HID, HID), lambda b: (0, 0)),
            pl.BlockSpec((1, HID), lambda b: (0, 0)),
            pl.BlockSpec((1, HID), lambda b: (0, 0)),
            pl.BlockSpec((1, NREL), lambda b: (0, 0)),
            pl.BlockSpec((HID, HID), lambda b: (0, 0)),
            pl.BlockSpec((1, HID), lambda b: (0, 0)),
            pl.BlockSpec((1, HID), lambda b: (0, 0)),
            pl.BlockSpec((1, NREL), lambda b: (0, 0)),
        ],
        out_specs=pl.BlockSpec((1, N, HID + EMB), lambda b: (b, 0, 0)),
    )

    return pl.pallas_call(
        _net_kernel,
        grid_spec=grid_spec,
        out_shape=jax.ShapeDtypeStruct((B, N, HID + EMB), jnp.float32),
    )(utterance_features, semantic_adj,
      W_fc1, row(b_fc1),
      W_gat0, row(a_src0), row(a_dst0), row(rel_bias0),
      W_gat1, row(a_src1), row(a_dst1), row(rel_bias1))
